# baseline (speedup 1.0000x reference)
import functools

import jax
import jax.numpy as jnp
from jax import lax
from jax.experimental import pallas as pl
from jax.experimental.pallas import tpu as pltpu
from jax.experimental.pallas import tpu_sc as plsc

_CH = 8
_NC, _NS = 2, 16
_NW = _NC * _NS


def _pack_body(x_ref, o_ref):
    xb = x_ref[0]
    C = xb.shape[-1]
    lo = lax.convert_element_type(
        lax.bitcast_convert_type(xb[:, : C // 2], jnp.uint16), jnp.uint32)
    hi = lax.convert_element_type(
        lax.bitcast_convert_type(xb[:, C // 2:], jnp.uint16), jnp.uint32)
    w = lax.bitcast_convert_type(lo | (hi << 16), jnp.int32)
    d = w.reshape(w.shape[0] // 2, 2, w.shape[1])
    o_ref[0] = jnp.concatenate([d[:, 0, :], d[:, 1, :]], axis=1)


def _tc_pack(xt, NTP=512):
    B, N, C = xt.shape
    return pl.pallas_call(
        _pack_body,
        grid=(B, N // NTP),
        in_specs=[pl.BlockSpec((1, NTP, C), lambda b, t: (b, t, 0))],
        out_specs=pl.BlockSpec((1, NTP // 2, C), lambda b, t: (b, t, 0)),
        out_shape=jax.ShapeDtypeStruct((B, N // 2, C), jnp.int32),
    )(xt)


def _make_sc_gather_max(B, N, C, K):
    C2 = C // 2
    rows_total = B * N
    assert rows_total % (_NW * _CH) == 0
    rows_per_w = rows_total // _NW
    assert N % rows_per_w == 0
    num_chunks = rows_per_w // _CH
    assert num_chunks % 2 == 0
    mesh = plsc.VectorSubcoreMesh(core_axis_name="c", subcore_axis_name="s")

    def body(xrows_hbm, e_hbm, out_hbm,
             idxj, idxi, xj0, xj1, xi0, xi1, out_v, sj0, sj1, si0, si1):
        wid = lax.axis_index("s") * _NC + lax.axis_index("c")
        row0 = wid * rows_per_w
        bidx = wid // (_NW // B)
        xb = xrows_hbm.at[bidx]

        pltpu.sync_copy(e_hbm.at[0, wid], idxj)
        pltpu.sync_copy(e_hbm.at[1, wid], idxi)

        bufs = ((xj0, xi0, sj0, si0), (xj1, xi1, sj1, si1))

        def start(t, bi):
            xj, xi, sj, si = bufs[bi]
            pltpu.async_copy(xb.at[idxj.at[t]], xj, sj)
            pltpu.async_copy(xb.at[idxi.at[t]], xi, si)

        def wait_buf(bi):
            xj, xi, sj, si = bufs[bi]
            pltpu.make_async_copy(xb.at[pl.ds(0, _CH * K)], xj, sj).wait()
            pltpu.make_async_copy(xb.at[pl.ds(0, _CH * K)], xi, si).wait()

        def compute(t, bi):
            xj, xi, _, _ = bufs[bi]

            def lo(w):
                return lax.bitcast_convert_type(w << 16, jnp.float32)

            def hi(w):
                return lax.bitcast_convert_type(w, jnp.float32)

            for r in range(_CH):
                base = r * K
                for cs in range(C2 // 16):
                    sl = pl.ds(cs * 16, 16)
                    wj = xj[base, sl]
                    wi = xi[base, sl]
                    mlo = lo(wj) - lo(wi)
                    mhi = hi(wj) - hi(wi)
                    for k in range(1, K):
                        wj = xj[base + k, sl]
                        wi = xi[base + k, sl]
                        mlo = jnp.maximum(mlo, lo(wj) - lo(wi))
                        mhi = jnp.maximum(mhi, hi(wj) - hi(wi))
                    out_v[r, pl.ds(cs * 16, 16)] = mlo
                    out_v[r, pl.ds(C2 + cs * 16, 16)] = mhi

            pltpu.sync_copy(out_v, out_hbm.at[pl.ds(row0 + t * _CH, _CH)])

        start(0, 0)

        @pl.loop(0, num_chunks, step=2)
        def _pipe(t):
            start(t + 1, 1)
            wait_buf(0)
            compute(t, 0)
            t2 = lax.select(t + 2 < num_chunks, t + 2, 0)
            start(t2, 0)
            wait_buf(1)
            compute(t + 1, 1)

        wait_buf(0)

    return pl.kernel(
        body,
        out_type=jax.ShapeDtypeStruct((rows_total, C), jnp.float32),
        mesh=mesh,
        compiler_params=pltpu.CompilerParams(use_tc_tiling_on_sc=False),
        scratch_types=[
            pltpu.VMEM((num_chunks, _CH * K), jnp.int32),
            pltpu.VMEM((num_chunks, _CH * K), jnp.int32),
            pltpu.VMEM((_CH * K, C2), jnp.int32),
            pltpu.VMEM((_CH * K, C2), jnp.int32),
            pltpu.VMEM((_CH * K, C2), jnp.int32),
            pltpu.VMEM((_CH * K, C2), jnp.int32),
            pltpu.VMEM((_CH, C), jnp.float32),
            pltpu.SemaphoreType.DMA,
            pltpu.SemaphoreType.DMA,
            pltpu.SemaphoreType.DMA,
            pltpu.SemaphoreType.DMA,
        ],
    )


def _mm_body(x_ref, xm_ref, we_ref, wo_ref, b_ref, o_ref):
    acc = lax.dot_general(
        x_ref[0], we_ref[...], (((1,), (1,)), ((), ())),
        preferred_element_type=jnp.float32)
    acc = acc + lax.dot_general(
        xm_ref[0], wo_ref[...], (((1,), (1,)), ((), ())),
        preferred_element_type=jnp.float32)
    acc = acc + b_ref[...]
    o_ref[0] = jnp.maximum(acc, 0.0)


def _tc_conv(xt, xmax3, W_e, W_o, bias_row, NT=2048):
    B, N, C = xt.shape
    OUT_C = W_e.shape[0]
    return pl.pallas_call(
        _mm_body,
        grid=(B, N // NT),
        in_specs=[
            pl.BlockSpec((1, NT, C), lambda b, t: (b, t, 0)),
            pl.BlockSpec((1, NT, C), lambda b, t: (b, t, 0)),
            pl.BlockSpec((OUT_C, C), lambda b, t: (0, 0)),
            pl.BlockSpec((OUT_C, C), lambda b, t: (0, 0)),
            pl.BlockSpec((1, OUT_C), lambda b, t: (0, 0)),
        ],
        out_specs=pl.BlockSpec((1, NT, OUT_C), lambda b, t: (b, t, 0)),
        out_shape=jax.ShapeDtypeStruct((B, N, OUT_C), jnp.float32),
    )(xt, xmax3, W_e, W_o, bias_row)


def kernel(x, edge_index, W, b):
    B, C, D, H, Wsp = x.shape
    n = D * H * Wsp
    K = edge_index.shape[-1]
    R = B * n

    xt = x.reshape(B, C, n).transpose(0, 2, 1)
    xt_bf = xt.astype(jnp.bfloat16)

    x_rows = _tc_pack(xt_bf).reshape(B, n, C // 2)

    rows_per_w = R // _NW
    num_chunks = rows_per_w // _CH
    e_all = edge_index.reshape(2, _NW, num_chunks, _CH * K)

    xmax = _make_sc_gather_max(B, n, C, K)(x_rows, e_all)

    W_e = W[:, 0::2].astype(jnp.bfloat16)
    W_o = W[:, 1::2].astype(jnp.bfloat16)
    xmax_bf = xmax.reshape(B, n, C).astype(jnp.bfloat16)
    out = _tc_conv(xt_bf, xmax_bf, W_e, W_o, b.reshape(1, -1))
    return out.transpose(0, 2, 1).reshape(B, W.shape[0], D, H, Wsp)


# speedup vs baseline: 1.0906x; 1.0906x over previous
import functools

import jax
import jax.numpy as jnp
from jax import lax
from jax.experimental import pallas as pl
from jax.experimental.pallas import tpu as pltpu
from jax.experimental.pallas import tpu_sc as plsc

_CH = 8
_NC, _NS = 2, 16
_NW = _NC * _NS


def _pack_body(x_ref, o_ref, obf_ref):
    xb = x_ref[0].astype(jnp.bfloat16)
    obf_ref[0] = xb
    C = xb.shape[-1]
    lo = lax.convert_element_type(
        lax.bitcast_convert_type(xb[:, : C // 2], jnp.uint16), jnp.uint32)
    hi = lax.convert_element_type(
        lax.bitcast_convert_type(xb[:, C // 2:], jnp.uint16), jnp.uint32)
    w = lax.bitcast_convert_type(lo | (hi << 16), jnp.int32)
    d = w.reshape(w.shape[0] // 2, 2, w.shape[1])
    o_ref[0, :, : C // 2] = d[:, 0, :]
    o_ref[0, :, C // 2:] = d[:, 1, :]


def _tc_pack(xt, NTP=1024):
    B, N, C = xt.shape
    return pl.pallas_call(
        _pack_body,
        grid=(B, N // NTP),
        in_specs=[pl.BlockSpec((1, NTP, C), lambda b, t: (b, t, 0))],
        out_specs=[
            pl.BlockSpec((1, NTP // 2, C), lambda b, t: (b, t, 0)),
            pl.BlockSpec((1, NTP, C), lambda b, t: (b, t, 0)),
        ],
        out_shape=[
            jax.ShapeDtypeStruct((B, N // 2, C), jnp.int32),
            jax.ShapeDtypeStruct((B, N, C), jnp.bfloat16),
        ],
    )(xt)


def _make_sc_gather_max(B, N, C, K):
    C2 = C // 2
    rows_total = B * N
    assert rows_total % (_NW * _CH) == 0
    rows_per_w = rows_total // _NW
    assert N % rows_per_w == 0
    num_chunks = rows_per_w // _CH
    assert num_chunks % 2 == 0
    mesh = plsc.VectorSubcoreMesh(core_axis_name="c", subcore_axis_name="s")

    def body(xrows_hbm, e_hbm, out_hbm,
             idxj, idxi, xj0, xj1, xi0, xi1, out_v, sj0, sj1, si0, si1):
        wid = lax.axis_index("s") * _NC + lax.axis_index("c")
        row0 = wid * rows_per_w
        bidx = wid // (_NW // B)
        xb = xrows_hbm.at[bidx]

        pltpu.sync_copy(e_hbm.at[0, wid], idxj)
        pltpu.sync_copy(e_hbm.at[1, wid], idxi)

        bufs = ((xj0, xi0, sj0, si0), (xj1, xi1, sj1, si1))

        def start(t, bi):
            xj, xi, sj, si = bufs[bi]
            pltpu.async_copy(xb.at[idxj.at[t]], xj, sj)
            pltpu.async_copy(xb.at[idxi.at[t]], xi, si)

        def wait_buf(bi):
            xj, xi, sj, si = bufs[bi]
            pltpu.make_async_copy(xb.at[pl.ds(0, _CH * K)], xj, sj).wait()
            pltpu.make_async_copy(xb.at[pl.ds(0, _CH * K)], xi, si).wait()

        def compute(t, bi):
            xj, xi, _, _ = bufs[bi]

            def lo(w):
                return lax.bitcast_convert_type(w << 16, jnp.float32)

            def hi(w):
                return lax.bitcast_convert_type(w, jnp.float32)

            for r in range(_CH):
                base = r * K
                for cs in range(C2 // 16):
                    sl = pl.ds(cs * 16, 16)
                    wj = xj[base, sl]
                    wi = xi[base, sl]
                    mlo = lo(wj) - lo(wi)
                    mhi = hi(wj) - hi(wi)
                    for k in range(1, K):
                        wj = xj[base + k, sl]
                        wi = xi[base + k, sl]
                        mlo = jnp.maximum(mlo, lo(wj) - lo(wi))
                        mhi = jnp.maximum(mhi, hi(wj) - hi(wi))
                    out_v[r, pl.ds(cs * 16, 16)] = mlo
                    out_v[r, pl.ds(C2 + cs * 16, 16)] = mhi

            pltpu.sync_copy(out_v, out_hbm.at[pl.ds(row0 + t * _CH, _CH)])

        start(0, 0)

        @pl.loop(0, num_chunks, step=2)
        def _pipe(t):
            start(t + 1, 1)
            wait_buf(0)
            compute(t, 0)
            t2 = lax.select(t + 2 < num_chunks, t + 2, 0)
            start(t2, 0)
            wait_buf(1)
            compute(t + 1, 1)

        wait_buf(0)

    return pl.kernel(
        body,
        out_type=jax.ShapeDtypeStruct((rows_total, C), jnp.float32),
        mesh=mesh,
        compiler_params=pltpu.CompilerParams(use_tc_tiling_on_sc=False),
        scratch_types=[
            pltpu.VMEM((num_chunks, _CH * K), jnp.int32),
            pltpu.VMEM((num_chunks, _CH * K), jnp.int32),
            pltpu.VMEM((_CH * K, C2), jnp.int32),
            pltpu.VMEM((_CH * K, C2), jnp.int32),
            pltpu.VMEM((_CH * K, C2), jnp.int32),
            pltpu.VMEM((_CH * K, C2), jnp.int32),
            pltpu.VMEM((_CH, C), jnp.float32),
            pltpu.SemaphoreType.DMA,
            pltpu.SemaphoreType.DMA,
            pltpu.SemaphoreType.DMA,
            pltpu.SemaphoreType.DMA,
        ],
    )


def _mm_body(x_ref, xm_ref, we_ref, wo_ref, b_ref, o_ref):
    acc = lax.dot_general(
        x_ref[0], we_ref[...], (((1,), (1,)), ((), ())),
        preferred_element_type=jnp.float32)
    acc = acc + lax.dot_general(
        xm_ref[0].astype(jnp.bfloat16), wo_ref[...], (((1,), (1,)), ((), ())),
        preferred_element_type=jnp.float32)
    acc = acc + b_ref[...]
    o_ref[0] = jnp.maximum(acc, 0.0)


def _tc_conv(xt, xmax3, W_e, W_o, bias_row, NT=2048):
    B, N, C = xt.shape
    OUT_C = W_e.shape[0]
    return pl.pallas_call(
        _mm_body,
        grid=(B, N // NT),
        in_specs=[
            pl.BlockSpec((1, NT, C), lambda b, t: (b, t, 0)),
            pl.BlockSpec((1, NT, C), lambda b, t: (b, t, 0)),
            pl.BlockSpec((OUT_C, C), lambda b, t: (0, 0)),
            pl.BlockSpec((OUT_C, C), lambda b, t: (0, 0)),
            pl.BlockSpec((1, OUT_C), lambda b, t: (0, 0)),
        ],
        out_specs=pl.BlockSpec((1, NT, OUT_C), lambda b, t: (b, t, 0)),
        out_shape=jax.ShapeDtypeStruct((B, N, OUT_C), jnp.float32),
    )(xt, xmax3, W_e, W_o, bias_row)


def kernel(x, edge_index, W, b):
    B, C, D, H, Wsp = x.shape
    n = D * H * Wsp
    K = edge_index.shape[-1]
    R = B * n

    xt = x.reshape(B, C, n).transpose(0, 2, 1)
    packed, xt_bf = _tc_pack(xt)
    x_rows = packed.reshape(B, n, C // 2)

    rows_per_w = R // _NW
    num_chunks = rows_per_w // _CH
    e_all = edge_index.reshape(2, _NW, num_chunks, _CH * K)

    xmax = _make_sc_gather_max(B, n, C, K)(x_rows, e_all)

    W_e = W[:, 0::2].astype(jnp.bfloat16)
    W_o = W[:, 1::2].astype(jnp.bfloat16)
    out = _tc_conv(xt_bf, xmax.reshape(B, n, C), W_e, W_o, b.reshape(1, -1))
    return out.transpose(0, 2, 1).reshape(B, W.shape[0], D, H, Wsp)


# speedup vs baseline: 1.1235x; 1.0302x over previous
import functools

import jax
import jax.numpy as jnp
from jax import lax
from jax.experimental import pallas as pl
from jax.experimental.pallas import tpu as pltpu
from jax.experimental.pallas import tpu_sc as plsc

_CH = 8
_NC, _NS = 2, 16
_NW = _NC * _NS


def _pack_body(x_ref, o_ref, obf_ref):
    xb = x_ref[0].astype(jnp.bfloat16)
    obf_ref[0] = xb
    C = xb.shape[-1]
    lo = lax.convert_element_type(
        lax.bitcast_convert_type(xb[:, : C // 2], jnp.uint16), jnp.uint32)
    hi = lax.convert_element_type(
        lax.bitcast_convert_type(xb[:, C // 2:], jnp.uint16), jnp.uint32)
    w = lax.bitcast_convert_type(lo | (hi << 16), jnp.int32)
    d = w.reshape(w.shape[0] // 2, 2, w.shape[1])
    o_ref[0, :, : C // 2] = d[:, 0, :]
    o_ref[0, :, C // 2:] = d[:, 1, :]


def _tc_pack(xt, NTP=1024):
    B, N, C = xt.shape
    return pl.pallas_call(
        _pack_body,
        grid=(B, N // NTP),
        in_specs=[pl.BlockSpec((1, NTP, C), lambda b, t: (b, t, 0))],
        out_specs=[
            pl.BlockSpec((1, NTP // 2, C), lambda b, t: (b, t, 0)),
            pl.BlockSpec((1, NTP, C), lambda b, t: (b, t, 0)),
        ],
        out_shape=[
            jax.ShapeDtypeStruct((B, N // 2, C), jnp.int32),
            jax.ShapeDtypeStruct((B, N, C), jnp.bfloat16),
        ],
    )(xt)


def _erepack_body(e_ref, o_ref):
    ek = e_ref[0, 0]
    et = jnp.transpose(ek)
    K = et.shape[1]
    d = et.reshape(et.shape[0] // 8, 8, K)
    o_ref[0, 0] = jnp.concatenate([d[:, i, :] for i in range(8)], axis=1)


def _tc_erepack(ep, NTE=2048):
    S, B, K, N = ep.shape
    return pl.pallas_call(
        _erepack_body,
        grid=(S, B, N // NTE),
        in_specs=[pl.BlockSpec((1, 1, K, NTE), lambda s, b, t: (s, b, 0, t))],
        out_specs=pl.BlockSpec((1, 1, NTE // 8, 8 * K),
                               lambda s, b, t: (s, b, t, 0)),
        out_shape=jax.ShapeDtypeStruct((S, B, N // 8, 8 * K), jnp.int32),
    )(ep)


def _make_sc_gather_max(B, N, C, K):
    C2 = C // 2
    rows_total = B * N
    assert rows_total % (_NW * _CH) == 0
    rows_per_w = rows_total // _NW
    assert N % rows_per_w == 0
    num_chunks = rows_per_w // _CH
    assert num_chunks % 2 == 0
    mesh = plsc.VectorSubcoreMesh(core_axis_name="c", subcore_axis_name="s")

    def body(xrows_hbm, e_hbm, out_hbm,
             idxj, idxi, xj0, xj1, xi0, xi1, out_v, sj0, sj1, si0, si1):
        wid = lax.axis_index("s") * _NC + lax.axis_index("c")
        row0 = wid * rows_per_w
        bidx = wid // (_NW // B)
        xb = xrows_hbm.at[bidx]

        lw = wid % (_NW // B)
        pltpu.sync_copy(e_hbm.at[0, bidx, pl.ds(lw * num_chunks, num_chunks)],
                        idxj)
        pltpu.sync_copy(e_hbm.at[1, bidx, pl.ds(lw * num_chunks, num_chunks)],
                        idxi)

        bufs = ((xj0, xi0, sj0, si0), (xj1, xi1, sj1, si1))

        def start(t, bi):
            xj, xi, sj, si = bufs[bi]
            pltpu.async_copy(xb.at[idxj.at[t]], xj, sj)
            pltpu.async_copy(xb.at[idxi.at[t]], xi, si)

        def wait_buf(bi):
            xj, xi, sj, si = bufs[bi]
            pltpu.make_async_copy(xb.at[pl.ds(0, _CH * K)], xj, sj).wait()
            pltpu.make_async_copy(xb.at[pl.ds(0, _CH * K)], xi, si).wait()

        def compute(t, bi):
            xj, xi, _, _ = bufs[bi]

            def lo(w):
                return lax.bitcast_convert_type(w << 16, jnp.float32)

            def hi(w):
                return lax.bitcast_convert_type(w, jnp.float32)

            for r in range(_CH):
                base = r * K
                for cs in range(C2 // 16):
                    sl = pl.ds(cs * 16, 16)
                    wj = xj[base, sl]
                    wi = xi[base, sl]
                    mlo = lo(wj) - lo(wi)
                    mhi = hi(wj) - hi(wi)
                    for k in range(1, K):
                        wj = xj[base + k, sl]
                        wi = xi[base + k, sl]
                        mlo = jnp.maximum(mlo, lo(wj) - lo(wi))
                        mhi = jnp.maximum(mhi, hi(wj) - hi(wi))
                    out_v[r, pl.ds(cs * 16, 16)] = mlo
                    out_v[r, pl.ds(C2 + cs * 16, 16)] = mhi

            pltpu.sync_copy(out_v, out_hbm.at[pl.ds(row0 + t * _CH, _CH)])

        start(0, 0)

        @pl.loop(0, num_chunks, step=2)
        def _pipe(t):
            start(t + 1, 1)
            wait_buf(0)
            compute(t, 0)
            t2 = lax.select(t + 2 < num_chunks, t + 2, 0)
            start(t2, 0)
            wait_buf(1)
            compute(t + 1, 1)

        wait_buf(0)

    return pl.kernel(
        body,
        out_type=jax.ShapeDtypeStruct((rows_total, C), jnp.float32),
        mesh=mesh,
        compiler_params=pltpu.CompilerParams(use_tc_tiling_on_sc=False),
        scratch_types=[
            pltpu.VMEM((num_chunks, _CH * K), jnp.int32),
            pltpu.VMEM((num_chunks, _CH * K), jnp.int32),
            pltpu.VMEM((_CH * K, C2), jnp.int32),
            pltpu.VMEM((_CH * K, C2), jnp.int32),
            pltpu.VMEM((_CH * K, C2), jnp.int32),
            pltpu.VMEM((_CH * K, C2), jnp.int32),
            pltpu.VMEM((_CH, C), jnp.float32),
            pltpu.SemaphoreType.DMA,
            pltpu.SemaphoreType.DMA,
            pltpu.SemaphoreType.DMA,
            pltpu.SemaphoreType.DMA,
        ],
    )


def _mm_body(x_ref, xm_ref, we_ref, wo_ref, b_ref, o_ref):
    acc = lax.dot_general(
        x_ref[0], we_ref[...], (((1,), (1,)), ((), ())),
        preferred_element_type=jnp.float32)
    acc = acc + lax.dot_general(
        xm_ref[0].astype(jnp.bfloat16), wo_ref[...], (((1,), (1,)), ((), ())),
        preferred_element_type=jnp.float32)
    acc = acc + b_ref[...]
    o_ref[0] = jnp.maximum(acc, 0.0)


def _tc_conv(xt, xmax3, W_e, W_o, bias_row, NT=2048):
    B, N, C = xt.shape
    OUT_C = W_e.shape[0]
    return pl.pallas_call(
        _mm_body,
        grid=(B, N // NT),
        in_specs=[
            pl.BlockSpec((1, NT, C), lambda b, t: (b, t, 0)),
            pl.BlockSpec((1, NT, C), lambda b, t: (b, t, 0)),
            pl.BlockSpec((OUT_C, C), lambda b, t: (0, 0)),
            pl.BlockSpec((OUT_C, C), lambda b, t: (0, 0)),
            pl.BlockSpec((1, OUT_C), lambda b, t: (0, 0)),
        ],
        out_specs=pl.BlockSpec((1, NT, OUT_C), lambda b, t: (b, t, 0)),
        out_shape=jax.ShapeDtypeStruct((B, N, OUT_C), jnp.float32),
    )(xt, xmax3, W_e, W_o, bias_row)


def kernel(x, edge_index, W, b):
    B, C, D, H, Wsp = x.shape
    n = D * H * Wsp
    K = edge_index.shape[-1]
    R = B * n

    xt = x.reshape(B, C, n).transpose(0, 2, 1)
    packed, xt_bf = _tc_pack(xt)
    x_rows = packed.reshape(B, n, C // 2)

    e_chunks = _tc_erepack(edge_index.transpose(0, 1, 3, 2))

    xmax = _make_sc_gather_max(B, n, C, K)(x_rows, e_chunks)

    W_e = W[:, 0::2].astype(jnp.bfloat16)
    W_o = W[:, 1::2].astype(jnp.bfloat16)
    out = _tc_conv(xt_bf, xmax.reshape(B, n, C), W_e, W_o, b.reshape(1, -1))
    return out.transpose(0, 2, 1).reshape(B, W.shape[0], D, H, Wsp)


# speedup vs baseline: 1.1714x; 1.0426x over previous
import functools

import jax
import jax.numpy as jnp
from jax import lax
from jax.experimental import pallas as pl
from jax.experimental.pallas import tpu as pltpu
from jax.experimental.pallas import tpu_sc as plsc

_CH = 8
_NC, _NS = 2, 16
_NW = _NC * _NS


def _pack_body(x_ref, o_ref, obf_ref):
    xb = x_ref[0].astype(jnp.bfloat16)
    obf_ref[0] = xb
    C = xb.shape[-1]
    lo = lax.convert_element_type(
        lax.bitcast_convert_type(xb[:, : C // 2], jnp.uint16), jnp.uint32)
    hi = lax.convert_element_type(
        lax.bitcast_convert_type(xb[:, C // 2:], jnp.uint16), jnp.uint32)
    w = lax.bitcast_convert_type(lo | (hi << 16), jnp.int32)
    d = w.reshape(w.shape[0] // 2, 2, w.shape[1])
    o_ref[0, :, : C // 2] = d[:, 0, :]
    o_ref[0, :, C // 2:] = d[:, 1, :]


def _tc_pack(xt, NTP=2048):
    B, N, C = xt.shape
    return pl.pallas_call(
        _pack_body,
        grid=(B, N // NTP),
        in_specs=[pl.BlockSpec((1, NTP, C), lambda b, t: (b, t, 0))],
        out_specs=[
            pl.BlockSpec((1, NTP // 2, C), lambda b, t: (b, t, 0)),
            pl.BlockSpec((1, NTP, C), lambda b, t: (b, t, 0)),
        ],
        out_shape=[
            jax.ShapeDtypeStruct((B, N // 2, C), jnp.int32),
            jax.ShapeDtypeStruct((B, N, C), jnp.bfloat16),
        ],
    )(xt)


def _erepack_body(e_ref, o_ref):
    ek = e_ref[0, 0]
    et = jnp.transpose(ek)
    K = et.shape[1]
    d = et.reshape(et.shape[0] // 8, 8, K)
    o_ref[0, 0] = jnp.concatenate([d[:, i, :] for i in range(8)], axis=1)


def _tc_erepack(ep, NTE=4096):
    S, B, K, N = ep.shape
    return pl.pallas_call(
        _erepack_body,
        grid=(S, B, N // NTE),
        in_specs=[pl.BlockSpec((1, 1, K, NTE), lambda s, b, t: (s, b, 0, t))],
        out_specs=pl.BlockSpec((1, 1, NTE // 8, 8 * K),
                               lambda s, b, t: (s, b, t, 0)),
        out_shape=jax.ShapeDtypeStruct((S, B, N // 8, 8 * K), jnp.int32),
    )(ep)


def _make_sc_gather_max(B, N, C, K):
    C2 = C // 2
    rows_total = B * N
    assert rows_total % (_NW * _CH) == 0
    rows_per_w = rows_total // _NW
    assert N % rows_per_w == 0
    num_chunks = rows_per_w // _CH
    assert num_chunks % 2 == 0
    mesh = plsc.VectorSubcoreMesh(core_axis_name="c", subcore_axis_name="s")

    def body(xrows_hbm, e_hbm, out_hbm,
             idxj, idxi, xj0, xj1, xi0, xi1, out_v, sj0, sj1, si0, si1):
        wid = lax.axis_index("s") * _NC + lax.axis_index("c")
        row0 = wid * rows_per_w
        bidx = wid // (_NW // B)
        xb = xrows_hbm.at[bidx]

        lw = wid % (_NW // B)
        pltpu.sync_copy(e_hbm.at[0, bidx, pl.ds(lw * num_chunks, num_chunks)],
                        idxj)
        pltpu.sync_copy(e_hbm.at[1, bidx, pl.ds(lw * num_chunks, num_chunks)],
                        idxi)

        bufs = ((xj0, xi0, sj0, si0), (xj1, xi1, sj1, si1))

        def start(t, bi):
            xj, xi, sj, si = bufs[bi]
            pltpu.async_copy(xb.at[idxj.at[t]], xj, sj)
            pltpu.async_copy(xb.at[idxi.at[t]], xi, si)

        def wait_buf(bi):
            xj, xi, sj, si = bufs[bi]
            pltpu.make_async_copy(xb.at[pl.ds(0, _CH * K)], xj, sj).wait()
            pltpu.make_async_copy(xb.at[pl.ds(0, _CH * K)], xi, si).wait()

        def compute(t, bi):
            xj, xi, _, _ = bufs[bi]

            def lo(w):
                return lax.bitcast_convert_type(w << 16, jnp.float32)

            def hi(w):
                return lax.bitcast_convert_type(w, jnp.float32)

            for r in range(_CH):
                base = r * K
                for cs in range(C2 // 16):
                    sl = pl.ds(cs * 16, 16)
                    wj = xj[base, sl]
                    wi = xi[base, sl]
                    mlo = lo(wj) - lo(wi)
                    mhi = hi(wj) - hi(wi)
                    for k in range(1, K):
                        wj = xj[base + k, sl]
                        wi = xi[base + k, sl]
                        mlo = jnp.maximum(mlo, lo(wj) - lo(wi))
                        mhi = jnp.maximum(mhi, hi(wj) - hi(wi))
                    out_v[r, pl.ds(cs * 16, 16)] = mlo
                    out_v[r, pl.ds(C2 + cs * 16, 16)] = mhi

            pltpu.sync_copy(out_v, out_hbm.at[pl.ds(row0 + t * _CH, _CH)])

        start(0, 0)

        @pl.loop(0, num_chunks, step=2)
        def _pipe(t):
            start(t + 1, 1)
            wait_buf(0)
            compute(t, 0)
            t2 = lax.select(t + 2 < num_chunks, t + 2, 0)
            start(t2, 0)
            wait_buf(1)
            compute(t + 1, 1)

        wait_buf(0)

    return pl.kernel(
        body,
        out_type=jax.ShapeDtypeStruct((rows_total, C), jnp.float32),
        mesh=mesh,
        compiler_params=pltpu.CompilerParams(use_tc_tiling_on_sc=False),
        scratch_types=[
            pltpu.VMEM((num_chunks, _CH * K), jnp.int32),
            pltpu.VMEM((num_chunks, _CH * K), jnp.int32),
            pltpu.VMEM((_CH * K, C2), jnp.int32),
            pltpu.VMEM((_CH * K, C2), jnp.int32),
            pltpu.VMEM((_CH * K, C2), jnp.int32),
            pltpu.VMEM((_CH * K, C2), jnp.int32),
            pltpu.VMEM((_CH, C), jnp.float32),
            pltpu.SemaphoreType.DMA,
            pltpu.SemaphoreType.DMA,
            pltpu.SemaphoreType.DMA,
            pltpu.SemaphoreType.DMA,
        ],
    )


def _mm_body(x_ref, xm_ref, we_ref, wo_ref, b_ref, o_ref):
    acc = lax.dot_general(
        x_ref[0], we_ref[...], (((1,), (1,)), ((), ())),
        preferred_element_type=jnp.float32)
    acc = acc + lax.dot_general(
        xm_ref[0].astype(jnp.bfloat16), wo_ref[...], (((1,), (1,)), ((), ())),
        preferred_element_type=jnp.float32)
    acc = acc + b_ref[...]
    o_ref[0] = jnp.maximum(acc, 0.0)


def _tc_conv(xt, xmax3, W_e, W_o, bias_row, NT=2048):
    B, N, C = xt.shape
    OUT_C = W_e.shape[0]
    return pl.pallas_call(
        _mm_body,
        grid=(B, N // NT),
        in_specs=[
            pl.BlockSpec((1, NT, C), lambda b, t: (b, t, 0)),
            pl.BlockSpec((1, NT, C), lambda b, t: (b, t, 0)),
            pl.BlockSpec((OUT_C, C), lambda b, t: (0, 0)),
            pl.BlockSpec((OUT_C, C), lambda b, t: (0, 0)),
            pl.BlockSpec((1, OUT_C), lambda b, t: (0, 0)),
        ],
        out_specs=pl.BlockSpec((1, NT, OUT_C), lambda b, t: (b, t, 0)),
        out_shape=jax.ShapeDtypeStruct((B, N, OUT_C), jnp.float32),
    )(xt, xmax3, W_e, W_o, bias_row)


def kernel(x, edge_index, W, b):
    B, C, D, H, Wsp = x.shape
    n = D * H * Wsp
    K = edge_index.shape[-1]
    R = B * n

    xt = x.reshape(B, C, n).transpose(0, 2, 1)
    packed, xt_bf = _tc_pack(xt)
    x_rows = packed.reshape(B, n, C // 2)

    e_chunks = _tc_erepack(edge_index.transpose(0, 1, 3, 2))

    xmax = _make_sc_gather_max(B, n, C, K)(x_rows, e_chunks)

    W_e = W[:, 0::2].astype(jnp.bfloat16)
    W_o = W[:, 1::2].astype(jnp.bfloat16)
    out = _tc_conv(xt_bf, xmax.reshape(B, n, C), W_e, W_o, b.reshape(1, -1))
    return out.transpose(0, 2, 1).reshape(B, W.shape[0], D, H, Wsp)


# speedup vs baseline: 1.1781x; 1.0057x over previous
import functools

import jax
import jax.numpy as jnp
from jax import lax
from jax.experimental import pallas as pl
from jax.experimental.pallas import tpu as pltpu
from jax.experimental.pallas import tpu_sc as plsc

_CH = 8
_NC, _NS = 2, 16
_NW = _NC * _NS


def _pack_body(x_ref, o_ref, obf_ref):
    xb = x_ref[0].astype(jnp.bfloat16)
    obf_ref[0] = xb
    C = xb.shape[-1]
    lo = lax.convert_element_type(
        lax.bitcast_convert_type(xb[:, : C // 2], jnp.uint16), jnp.uint32)
    hi = lax.convert_element_type(
        lax.bitcast_convert_type(xb[:, C // 2:], jnp.uint16), jnp.uint32)
    w = lax.bitcast_convert_type(lo | (hi << 16), jnp.int32)
    d = w.reshape(w.shape[0] // 2, 2, w.shape[1])
    o_ref[0, :, : C // 2] = d[:, 0, :]
    o_ref[0, :, C // 2:] = d[:, 1, :]


def _tc_pack(xt, NTP=2048):
    B, N, C = xt.shape
    return pl.pallas_call(
        _pack_body,
        grid=(B, N // NTP),
        in_specs=[pl.BlockSpec((1, NTP, C), lambda b, t: (b, t, 0))],
        out_specs=[
            pl.BlockSpec((1, NTP // 2, C), lambda b, t: (b, t, 0)),
            pl.BlockSpec((1, NTP, C), lambda b, t: (b, t, 0)),
        ],
        out_shape=[
            jax.ShapeDtypeStruct((B, N // 2, C), jnp.int32),
            jax.ShapeDtypeStruct((B, N, C), jnp.bfloat16),
        ],
    )(xt)


def _erepack_body(e_ref, o_ref):
    ek = e_ref[0, 0]
    et = jnp.transpose(ek)
    K = et.shape[1]
    d = et.reshape(et.shape[0] // 8, 8, K)
    o_ref[0, 0] = jnp.concatenate([d[:, i, :] for i in range(8)], axis=1)


def _tc_erepack(ep, NTE=8192):
    S, B, K, N = ep.shape
    return pl.pallas_call(
        _erepack_body,
        grid=(S, B, N // NTE),
        in_specs=[pl.BlockSpec((1, 1, K, NTE), lambda s, b, t: (s, b, 0, t))],
        out_specs=pl.BlockSpec((1, 1, NTE // 8, 8 * K),
                               lambda s, b, t: (s, b, t, 0)),
        out_shape=jax.ShapeDtypeStruct((S, B, N // 8, 8 * K), jnp.int32),
    )(ep)


def _make_sc_gather_max(B, N, C, K):
    C2 = C // 2
    rows_total = B * N
    assert rows_total % (_NW * _CH) == 0
    rows_per_w = rows_total // _NW
    assert N % rows_per_w == 0
    num_chunks = rows_per_w // _CH
    assert num_chunks % 2 == 0
    mesh = plsc.VectorSubcoreMesh(core_axis_name="c", subcore_axis_name="s")

    def body(xrows_hbm, e_hbm, out_hbm,
             idxj, idxi, xj0, xj1, xi0, xi1, out_v, sj0, sj1, si0, si1):
        wid = lax.axis_index("s") * _NC + lax.axis_index("c")
        row0 = wid * rows_per_w
        bidx = wid // (_NW // B)
        xb = xrows_hbm.at[bidx]

        lw = wid % (_NW // B)
        pltpu.sync_copy(e_hbm.at[0, bidx, pl.ds(lw * num_chunks, num_chunks)],
                        idxj)
        pltpu.sync_copy(e_hbm.at[1, bidx, pl.ds(lw * num_chunks, num_chunks)],
                        idxi)

        bufs = ((xj0, xi0, sj0, si0), (xj1, xi1, sj1, si1))

        def start(t, bi):
            xj, xi, sj, si = bufs[bi]
            pltpu.async_copy(xb.at[idxj.at[t]], xj, sj)
            pltpu.async_copy(xb.at[idxi.at[t]], xi, si)

        def wait_buf(bi):
            xj, xi, sj, si = bufs[bi]
            pltpu.make_async_copy(xb.at[pl.ds(0, _CH * K)], xj, sj).wait()
            pltpu.make_async_copy(xb.at[pl.ds(0, _CH * K)], xi, si).wait()

        def compute(t, bi):
            xj, xi, _, _ = bufs[bi]

            def lo(w):
                return lax.bitcast_convert_type(w << 16, jnp.float32)

            def hi(w):
                return lax.bitcast_convert_type(w, jnp.float32)

            for r in range(_CH):
                base = r * K
                for cs in range(C2 // 16):
                    sl = pl.ds(cs * 16, 16)
                    wj = xj[base, sl]
                    wi = xi[base, sl]
                    mlo = lo(wj) - lo(wi)
                    mhi = hi(wj) - hi(wi)
                    for k in range(1, K):
                        wj = xj[base + k, sl]
                        wi = xi[base + k, sl]
                        mlo = jnp.maximum(mlo, lo(wj) - lo(wi))
                        mhi = jnp.maximum(mhi, hi(wj) - hi(wi))
                    out_v[r, pl.ds(cs * 16, 16)] = mlo
                    out_v[r, pl.ds(C2 + cs * 16, 16)] = mhi

            pltpu.sync_copy(out_v, out_hbm.at[pl.ds(row0 + t * _CH, _CH)])

        start(0, 0)

        @pl.loop(0, num_chunks, step=2)
        def _pipe(t):
            start(t + 1, 1)
            wait_buf(0)
            compute(t, 0)
            t2 = lax.select(t + 2 < num_chunks, t + 2, 0)
            start(t2, 0)
            wait_buf(1)
            compute(t + 1, 1)

        wait_buf(0)

    return pl.kernel(
        body,
        out_type=jax.ShapeDtypeStruct((rows_total, C), jnp.float32),
        mesh=mesh,
        compiler_params=pltpu.CompilerParams(use_tc_tiling_on_sc=False),
        scratch_types=[
            pltpu.VMEM((num_chunks, _CH * K), jnp.int32),
            pltpu.VMEM((num_chunks, _CH * K), jnp.int32),
            pltpu.VMEM((_CH * K, C2), jnp.int32),
            pltpu.VMEM((_CH * K, C2), jnp.int32),
            pltpu.VMEM((_CH * K, C2), jnp.int32),
            pltpu.VMEM((_CH * K, C2), jnp.int32),
            pltpu.VMEM((_CH, C), jnp.float32),
            pltpu.SemaphoreType.DMA,
            pltpu.SemaphoreType.DMA,
            pltpu.SemaphoreType.DMA,
            pltpu.SemaphoreType.DMA,
        ],
    )


def _mm_body(x_ref, xm_ref, we_ref, wo_ref, b_ref, o_ref):
    acc = lax.dot_general(
        x_ref[0], we_ref[...], (((1,), (1,)), ((), ())),
        preferred_element_type=jnp.float32)
    acc = acc + lax.dot_general(
        xm_ref[0].astype(jnp.bfloat16), wo_ref[...], (((1,), (1,)), ((), ())),
        preferred_element_type=jnp.float32)
    acc = acc + b_ref[...]
    o_ref[0] = jnp.maximum(acc, 0.0)


def _tc_conv(xt, xmax3, W_e, W_o, bias_row, NT=4096):
    B, N, C = xt.shape
    OUT_C = W_e.shape[0]
    return pl.pallas_call(
        _mm_body,
        grid=(B, N // NT),
        in_specs=[
            pl.BlockSpec((1, NT, C), lambda b, t: (b, t, 0)),
            pl.BlockSpec((1, NT, C), lambda b, t: (b, t, 0)),
            pl.BlockSpec((OUT_C, C), lambda b, t: (0, 0)),
            pl.BlockSpec((OUT_C, C), lambda b, t: (0, 0)),
            pl.BlockSpec((1, OUT_C), lambda b, t: (0, 0)),
        ],
        out_specs=pl.BlockSpec((1, NT, OUT_C), lambda b, t: (b, t, 0)),
        out_shape=jax.ShapeDtypeStruct((B, N, OUT_C), jnp.float32),
    )(xt, xmax3, W_e, W_o, bias_row)


def kernel(x, edge_index, W, b):
    B, C, D, H, Wsp = x.shape
    n = D * H * Wsp
    K = edge_index.shape[-1]
    R = B * n

    xt = x.reshape(B, C, n).transpose(0, 2, 1)
    packed, xt_bf = _tc_pack(xt)
    x_rows = packed.reshape(B, n, C // 2)

    e_chunks = _tc_erepack(edge_index.transpose(0, 1, 3, 2))

    xmax = _make_sc_gather_max(B, n, C, K)(x_rows, e_chunks)

    W_e = W[:, 0::2].astype(jnp.bfloat16)
    W_o = W[:, 1::2].astype(jnp.bfloat16)
    out = _tc_conv(xt_bf, xmax.reshape(B, n, C), W_e, W_o, b.reshape(1, -1))
    return out.transpose(0, 2, 1).reshape(B, W.shape[0], D, H, Wsp)
